# baseline (speedup 1.0000x reference)
import jax
import jax.numpy as jnp
import numpy as np
from jax import lax
from jax.experimental import pallas as pl
from jax.experimental.pallas import tpu as pltpu
from jax.experimental.pallas import tpu_sc as plsc

NPOINT = 10
L = 16
NC = 2
NS = 16
TECS_PER_BATCH = 4

NEG_BIG = np.float32(-3e38)
BIG_IDX = np.int32(1 << 30)
UNROLL = 8


def _build(B, N):
    C = N // TECS_PER_BATCH
    STEPS = C // L
    GROUPS = (B * TECS_PER_BATCH) // (NC * NS)
    BPCG = NS // TECS_PER_BATCH

    def body(xyz_hbm, out_hbm, raw0, raw1, db, rowbuf, buf4, outbuf,
             shared, dsem):
        c = lax.axis_index("c")
        s = lax.axis_index("s")
        lanes = lax.iota(jnp.int32, L)
        lanes3 = lanes * 3
        coordsel = jnp.minimum(lanes, 2)
        q = s % TECS_PER_BATCH
        qbase = (s // TECS_PER_BATCH) * TECS_PER_BATCH
        p0 = q * C
        raws = [raw0, raw1]

        def splat_f(v):
            return jnp.full((L,), v, dtype=jnp.float32)

        def batch_of(g):
            return c * (B // NC) + g * BPCG + s // TECS_PER_BATCH

        def raw_src(g):
            return xyz_hbm.at[pl.ds((batch_of(g) * N + p0) * 3, 3 * C)]

        pltpu.sync_copy(raw_src(0), raw0)
        if GROUPS > 1:
            prefetch = pltpu.async_copy(raw_src(1), raw1, dsem)

        for g in range(GROUPS):
            rb = raws[g % 2]
            if g > 0:
                prefetch.wait()

            def exchange(row, slot):
                rowbuf[...] = row
                off = slot * NS * L
                pltpu.sync_copy(rowbuf, shared.at[pl.ds(off + s * L, L)])
                plsc.subcore_barrier()
                pltpu.sync_copy(
                    shared.at[pl.ds(off + qbase * L, TECS_PER_BATCH * L)],
                    buf4)
                r = buf4[pl.ds(0, L)]
                wv, wi, wx, wy, wz = r[0], r[1], r[2], r[3], r[4]
                for j in range(1, TECS_PER_BATCH):
                    r = buf4[pl.ds(j * L, L)]
                    v, i = r[0], r[1]
                    better = jnp.logical_or(
                        v > wv, jnp.logical_and(v == wv, i < wi))
                    wv = jnp.where(better, v, wv)
                    wi = jnp.where(better, i, wi)
                    wx = jnp.where(better, r[2], wx)
                    wy = jnp.where(better, r[3], wy)
                    wz = jnp.where(better, r[4], wz)
                return wi, wx, wy, wz

            def make_row(val_v, idx_v, xv, yv, zv):
                return jnp.where(lanes == 0, val_v,
                       jnp.where(lanes == 1, idx_v,
                       jnp.where(lanes == 2, xv,
                       jnp.where(lanes == 3, yv,
                       jnp.where(lanes == 4, zv, splat_f(0.0))))))

            r0 = rb[pl.ds(0, L)]
            x0, y0, z0 = r0[0], r0[1], r0[2]
            val_s = jnp.where(q == 0, np.float32(3e38), NEG_BIG)
            idx_s = jnp.where(q == 0, np.float32(0.0), np.float32(2.0**30))
            wi, cx, cy, cz = exchange(
                make_row(splat_f(val_s), splat_f(idx_s),
                         splat_f(x0), splat_f(y0), splat_f(z0)), 0)
            outvec = jnp.where(lanes == 0, wi.astype(jnp.int32),
                               jnp.zeros((L,), jnp.int32))

            for k in range(NPOINT - 1):
                cxv = splat_f(cx)
                cyv = splat_f(cy)
                czv = splat_f(cz)

                @plsc.parallel_loop(0, STEPS, step=1, unroll=UNROLL,
                                    carry=(splat_f(NEG_BIG),
                                           jnp.zeros((L,), jnp.int32)))
                def pass_carry(i, carry):
                    bestv, besti = carry
                    o = i * L
                    ix = lanes3 + 3 * o
                    dx = plsc.load_gather(rb, [ix]) - cxv
                    dy = plsc.load_gather(rb, [ix + 1]) - cyv
                    dz = plsc.load_gather(rb, [ix + 2]) - czv
                    d = dx * dx + dy * dy + dz * dz
                    if k == 0:
                        nd = d
                    else:
                        nd = jnp.minimum(db[pl.ds(o, L)], d)
                    if k < NPOINT - 2:
                        db[pl.ds(o, L)] = nd
                    m = nd > bestv
                    bestv = jnp.where(m, nd, bestv)
                    besti = jnp.where(m, lanes + o, besti)
                    return bestv, besti

                bestv, besti = pass_carry

                mx = jnp.max(bestv)
                cand = jnp.where(bestv == mx, besti, BIG_IDX)
                bi = jnp.min(cand)
                bsel = jnp.full((L,), 3 * bi, jnp.int32) + coordsel
                rwin = plsc.load_gather(rb, [bsel])
                bx, by, bz = rwin[0], rwin[1], rwin[2]
                gidx_f = (bi + p0).astype(jnp.float32)
                wi, cx, cy, cz = exchange(
                    make_row(splat_f(mx), splat_f(gidx_f),
                             splat_f(bx), splat_f(by), splat_f(bz)),
                    (k + 1) % 2)
                outvec = jnp.where(lanes == k + 1, wi.astype(jnp.int32),
                                   outvec)

            outbuf[...] = outvec

            @pl.when(q == 0)
            def _():
                pltpu.sync_copy(outbuf, out_hbm.at[pl.ds(batch_of(g) * L, L)])

    return pl.kernel(
        body,
        out_type=jax.ShapeDtypeStruct((B * L,), jnp.int32),
        mesh=plsc.VectorSubcoreMesh(core_axis_name="c", subcore_axis_name="s",
                                    num_cores=NC, num_subcores=NS),
        compiler_params=pltpu.CompilerParams(needs_layout_passes=False,
                                             use_tc_tiling_on_sc=False),
        scratch_types=[
            pltpu.VMEM((3 * C,), jnp.float32),
            pltpu.VMEM((3 * C,), jnp.float32),
            pltpu.VMEM((C,), jnp.float32),
            pltpu.VMEM((L,), jnp.float32),
            pltpu.VMEM((TECS_PER_BATCH * L,), jnp.float32),
            pltpu.VMEM((L,), jnp.int32),
            pltpu.VMEM_SHARED((2 * NS * L,), jnp.float32),
            pltpu.SemaphoreType.DMA,
        ],
    )


@jax.jit
def kernel(xyz):
    B, N, _ = xyz.shape
    out = _build(B, N)(xyz.reshape(B * N * 3))
    return out.reshape(B, L)[:, :NPOINT]


# speedup vs baseline: 1.0044x; 1.0044x over previous
import jax
import jax.numpy as jnp
import numpy as np
from jax import lax
from jax.experimental import pallas as pl
from jax.experimental.pallas import tpu as pltpu
from jax.experimental.pallas import tpu_sc as plsc

NPOINT = 10
L = 16
NC = 2
NS = 16
TECS_PER_BATCH = 4

NEG_BIG = np.float32(-3e38)
BIG_IDX = np.int32(1 << 30)
UNROLL = 8


def _build(B, N):
    C = N // TECS_PER_BATCH
    STEPS = C // L
    GROUPS = (B * TECS_PER_BATCH) // (NC * NS)
    BPCG = NS // TECS_PER_BATCH

    def body(xyz_hbm, out_hbm, raw, xb, yb, zb, db, rowbuf, buf4, outbuf,
             shared):
        c = lax.axis_index("c")
        s = lax.axis_index("s")
        lanes = lax.iota(jnp.int32, L)
        q = s % TECS_PER_BATCH
        qbase = (s // TECS_PER_BATCH) * TECS_PER_BATCH
        p0 = q * C

        def splat_f(v):
            return jnp.full((L,), v, dtype=jnp.float32)

        def exchange(row):
            rowbuf[...] = row
            pltpu.sync_copy(rowbuf, shared.at[pl.ds(s * L, L)])
            plsc.subcore_barrier()
            pltpu.sync_copy(shared.at[pl.ds(qbase * L, TECS_PER_BATCH * L)],
                            buf4)
            plsc.subcore_barrier()
            r = buf4[pl.ds(0, L)]
            wv, wi, wx, wy, wz = r[0], r[1], r[2], r[3], r[4]
            for j in range(1, TECS_PER_BATCH):
                r = buf4[pl.ds(j * L, L)]
                v, i = r[0], r[1]
                better = jnp.logical_or(v > wv,
                                        jnp.logical_and(v == wv, i < wi))
                wv = jnp.where(better, v, wv)
                wi = jnp.where(better, i, wi)
                wx = jnp.where(better, r[2], wx)
                wy = jnp.where(better, r[3], wy)
                wz = jnp.where(better, r[4], wz)
            return wi, wx, wy, wz

        def make_row(val_v, idx_v, xv, yv, zv):
            return jnp.where(lanes == 0, val_v,
                   jnp.where(lanes == 1, idx_v,
                   jnp.where(lanes == 2, xv,
                   jnp.where(lanes == 3, yv,
                   jnp.where(lanes == 4, zv, splat_f(0.0))))))

        perm = []
        for t in range(3):
            pos = lanes * 3 + t
            perm.append((pos % 16, pos // 16))

        for g in range(GROUPS):
            b = c * (B // NC) + g * BPCG + s // TECS_PER_BATCH

            pltpu.sync_copy(xyz_hbm.at[pl.ds((b * N + p0) * 3, 3 * C)], raw)

            @plsc.parallel_loop(0, STEPS, step=1, unroll=4)
            def deint(i):
                o = i * L
                ro = 3 * o
                win = [raw[pl.ds(ro, L)], raw[pl.ds(ro + L, L)],
                       raw[pl.ds(ro + 2 * L, L)]]
                for t, dst in ((0, xb), (1, yb), (2, zb)):
                    idxv, srcv = perm[t]
                    picks = [w.at[idxv].get(mode="promise_in_bounds")
                             for w in win]
                    dst[pl.ds(o, L)] = jnp.where(
                        srcv == 0, picks[0],
                        jnp.where(srcv == 1, picks[1], picks[2]))

            x0 = xb[pl.ds(0, L)][0]
            y0 = yb[pl.ds(0, L)][0]
            z0 = zb[pl.ds(0, L)][0]
            val_s = jnp.where(q == 0, np.float32(3e38), NEG_BIG)
            idx_s = jnp.where(q == 0, np.float32(0.0), np.float32(2.0**30))
            wi, cx, cy, cz = exchange(
                make_row(splat_f(val_s), splat_f(idx_s),
                         splat_f(x0), splat_f(y0), splat_f(z0)))
            outvec = jnp.where(lanes == 0, wi.astype(jnp.int32),
                               jnp.zeros((L,), jnp.int32))

            for k in range(NPOINT - 1):
                cxv = splat_f(cx)
                cyv = splat_f(cy)
                czv = splat_f(cz)

                @plsc.parallel_loop(0, STEPS, step=1, unroll=UNROLL,
                                    carry=(splat_f(NEG_BIG),
                                           jnp.zeros((L,), jnp.int32)))
                def pass_carry(i, carry):
                    bestv, besti = carry
                    o = i * L
                    dx = xb[pl.ds(o, L)] - cxv
                    dy = yb[pl.ds(o, L)] - cyv
                    dz = zb[pl.ds(o, L)] - czv
                    d = dx * dx + dy * dy + dz * dz
                    if k == 0:
                        nd = d
                    else:
                        nd = jnp.minimum(db[pl.ds(o, L)], d)
                    db[pl.ds(o, L)] = nd
                    m = nd > bestv
                    bestv = jnp.where(m, nd, bestv)
                    besti = jnp.where(m, lanes + o, besti)
                    return bestv, besti

                bestv, besti = pass_carry

                mx = jnp.max(bestv)
                cand = jnp.where(bestv == mx, besti, BIG_IDX)
                bi = jnp.min(cand)
                o = (bi // L) * L
                sel = lanes == (bi - o)
                bx = jnp.sum(jnp.where(sel, xb[pl.ds(o, L)], 0.0))
                by = jnp.sum(jnp.where(sel, yb[pl.ds(o, L)], 0.0))
                bz = jnp.sum(jnp.where(sel, zb[pl.ds(o, L)], 0.0))
                gidx_f = (bi + p0).astype(jnp.float32)
                wi, cx, cy, cz = exchange(
                    make_row(splat_f(mx), splat_f(gidx_f),
                             splat_f(bx), splat_f(by), splat_f(bz)))
                outvec = jnp.where(lanes == k + 1, wi.astype(jnp.int32),
                                   outvec)

            outbuf[...] = outvec

            @pl.when(q == 0)
            def _():
                pltpu.sync_copy(outbuf, out_hbm.at[pl.ds(b * L, L)])

    return pl.kernel(
        body,
        out_type=jax.ShapeDtypeStruct((B * L,), jnp.int32),
        mesh=plsc.VectorSubcoreMesh(core_axis_name="c", subcore_axis_name="s",
                                    num_cores=NC, num_subcores=NS),
        compiler_params=pltpu.CompilerParams(needs_layout_passes=False, use_tc_tiling_on_sc=False, disable_bounds_checks=True),
        scratch_types=[
            pltpu.VMEM((3 * C,), jnp.float32),
            pltpu.VMEM((C,), jnp.float32),
            pltpu.VMEM((C,), jnp.float32),
            pltpu.VMEM((C,), jnp.float32),
            pltpu.VMEM((C,), jnp.float32),
            pltpu.VMEM((L,), jnp.float32),
            pltpu.VMEM((TECS_PER_BATCH * L,), jnp.float32),
            pltpu.VMEM((L,), jnp.int32),
            pltpu.VMEM_SHARED((NS * L,), jnp.float32),
        ],
    )


@jax.jit
def kernel(xyz):
    B, N, _ = xyz.shape
    out = _build(B, N)(xyz.reshape(B * N * 3))
    return out.reshape(B, L)[:, :NPOINT]


# speedup vs baseline: 43.5719x; 43.3819x over previous
import jax
import jax.numpy as jnp
import numpy as np
from jax import lax
from jax.experimental import pallas as pl
from jax.experimental.pallas import tpu as pltpu
from jax.experimental.pallas import tpu_sc as plsc

NPOINT = 10
L = 16
NC = 2
NS = 16
TECS_PER_BATCH = 4

NEG_BIG = np.float32(-3e38)
BIG_IDX = np.int32(1 << 30)
UNROLL = 4


def _build(B, N):
    C = N // TECS_PER_BATCH
    STEPS = C // L
    GROUPS = (B * TECS_PER_BATCH) // (NC * NS)
    BPCG = NS // TECS_PER_BATCH

    def body(xt_hbm, out_hbm, xb0, yb0, zb0, xb1, yb1, zb1, db, rowbuf,
             buf4, outbuf, shared, dsem):
        c = lax.axis_index("c")
        s = lax.axis_index("s")
        lanes = lax.iota(jnp.int32, L)
        q = s % TECS_PER_BATCH
        qbase = (s // TECS_PER_BATCH) * TECS_PER_BATCH
        p0 = q * C

        def splat_f(v):
            return jnp.full((L,), v, dtype=jnp.float32)

        def exchange(row, slot):
            rowbuf[...] = row
            off = slot * NS * L
            pltpu.sync_copy(rowbuf, shared.at[pl.ds(off + s * L, L)])
            plsc.subcore_barrier()
            pltpu.sync_copy(
                shared.at[pl.ds(off + qbase * L, TECS_PER_BATCH * L)], buf4)
            r = buf4[pl.ds(0, L)]
            wv, wi, wx, wy, wz = r[0], r[1], r[2], r[3], r[4]
            for j in range(1, TECS_PER_BATCH):
                r = buf4[pl.ds(j * L, L)]
                v, i = r[0], r[1]
                better = jnp.logical_or(v > wv,
                                        jnp.logical_and(v == wv, i < wi))
                wv = jnp.where(better, v, wv)
                wi = jnp.where(better, i, wi)
                wx = jnp.where(better, r[2], wx)
                wy = jnp.where(better, r[3], wy)
                wz = jnp.where(better, r[4], wz)
            return wi, wx, wy, wz

        def make_row(val_v, idx_v, xv, yv, zv):
            return jnp.where(lanes == 0, val_v,
                   jnp.where(lanes == 1, idx_v,
                   jnp.where(lanes == 2, xv,
                   jnp.where(lanes == 3, yv,
                   jnp.where(lanes == 4, zv, splat_f(0.0))))))

        def plane_src(g, t):
            b = c * (B // NC) + g * BPCG + s // TECS_PER_BATCH
            return xt_hbm.at[pl.ds(t, 1), pl.ds(b, 1), pl.ds(p0, C)]

        planes = [(xb0, yb0, zb0), (xb1, yb1, zb1)]

        st = [pltpu.async_copy(plane_src(0, t), planes[0][t], dsem)
              for t in range(3)]
        for h in st:
            h.wait()
        pf = [pltpu.async_copy(plane_src(1, t), planes[1][t], dsem)
              for t in range(3)] if GROUPS > 1 else []

        for g in range(GROUPS):
            b = c * (B // NC) + g * BPCG + s // TECS_PER_BATCH
            xb, yb, zb = planes[g % 2]
            if g > 0:
                for h in pf:
                    h.wait()

            x0 = xb[0, 0, pl.ds(0, L)][0]
            y0 = yb[0, 0, pl.ds(0, L)][0]
            z0 = zb[0, 0, pl.ds(0, L)][0]
            val_s = jnp.where(q == 0, np.float32(3e38), NEG_BIG)
            idx_s = jnp.where(q == 0, np.float32(0.0), np.float32(2.0**30))
            wi, cx, cy, cz = exchange(
                make_row(splat_f(val_s), splat_f(idx_s),
                         splat_f(x0), splat_f(y0), splat_f(z0)), 0)
            outvec = jnp.where(lanes == 0, wi.astype(jnp.int32),
                               jnp.zeros((L,), jnp.int32))

            for k in range(NPOINT - 1):
                cxv = splat_f(cx)
                cyv = splat_f(cy)
                czv = splat_f(cz)

                @plsc.parallel_loop(0, STEPS, step=1, unroll=UNROLL,
                                    carry=(splat_f(NEG_BIG),
                                           jnp.zeros((L,), jnp.int32)))
                def pass_carry(i, carry):
                    bestv, besti = carry
                    o = i * L
                    dx = xb[0, 0, pl.ds(o, L)] - cxv
                    dy = yb[0, 0, pl.ds(o, L)] - cyv
                    dz = zb[0, 0, pl.ds(o, L)] - czv
                    d = dx * dx + dy * dy + dz * dz
                    if k == 0:
                        nd = d
                    else:
                        nd = jnp.minimum(db[pl.ds(o, L)], d)
                    if k < NPOINT - 2:
                        db[pl.ds(o, L)] = nd
                    m = nd > bestv
                    bestv = jnp.where(m, nd, bestv)
                    besti = jnp.where(m, lanes + o, besti)
                    return bestv, besti

                bestv, besti = pass_carry

                mx = jnp.max(bestv)
                cand = jnp.where(bestv == mx, besti, BIG_IDX)
                bi = jnp.min(cand)
                o = (bi // L) * L
                sel = lanes == (bi - o)
                bx = jnp.sum(jnp.where(sel, xb[0, 0, pl.ds(o, L)], 0.0))
                by = jnp.sum(jnp.where(sel, yb[0, 0, pl.ds(o, L)], 0.0))
                bz = jnp.sum(jnp.where(sel, zb[0, 0, pl.ds(o, L)], 0.0))
                gidx_f = (bi + p0).astype(jnp.float32)
                wi, cx, cy, cz = exchange(
                    make_row(splat_f(mx), splat_f(gidx_f),
                             splat_f(bx), splat_f(by), splat_f(bz)),
                    (k + 1) % 2)
                outvec = jnp.where(lanes == k + 1, wi.astype(jnp.int32),
                                   outvec)

            outbuf[...] = outvec

            @pl.when(q == 0)
            def _():
                pltpu.sync_copy(outbuf, out_hbm.at[pl.ds(b * L, L)])

    return pl.kernel(
        body,
        out_type=jax.ShapeDtypeStruct((B * L,), jnp.int32),
        mesh=plsc.VectorSubcoreMesh(core_axis_name="c", subcore_axis_name="s",
                                    num_cores=NC, num_subcores=NS),
        compiler_params=pltpu.CompilerParams(needs_layout_passes=False,
                                             use_tc_tiling_on_sc=True,
                                             skip_device_barrier=True),
        scratch_types=[
            pltpu.VMEM((1, 1, C), jnp.float32),
            pltpu.VMEM((1, 1, C), jnp.float32),
            pltpu.VMEM((1, 1, C), jnp.float32),
            pltpu.VMEM((1, 1, C), jnp.float32),
            pltpu.VMEM((1, 1, C), jnp.float32),
            pltpu.VMEM((1, 1, C), jnp.float32),
            pltpu.VMEM((C,), jnp.float32),
            pltpu.VMEM((L,), jnp.float32),
            pltpu.VMEM((TECS_PER_BATCH * L,), jnp.float32),
            pltpu.VMEM((L,), jnp.int32),
            pltpu.VMEM_SHARED((2 * NS * L,), jnp.float32),
            pltpu.SemaphoreType.DMA,
        ],
    )


@jax.jit
def kernel(xyz):
    B, N, _ = xyz.shape
    out = _build(B, N)(jnp.transpose(xyz, (2, 0, 1)))
    return out.reshape(B, L)[:, :NPOINT]
